# baseline (speedup 1.0000x reference)
import functools

import jax
import jax.numpy as jnp
from jax import lax
from jax.experimental import pallas as pl
from jax.experimental.pallas import tpu as pltpu
from jax.experimental.pallas import tpu_sc as plsc

D = 128
D2 = D // 2
K = 32
NC = 2
NS = 16
NW = NC * NS
CPW = D2 // NW
L = 16
CH = 1024


def _matmul_body(x_ref, wt_ref, wd_ref, hs_ref, hd_ref):
    x = x_ref[...]
    hs = jnp.dot(x, wt_ref[...], preferred_element_type=jnp.float32)
    hd = jnp.dot(x, wd_ref[...], preferred_element_type=jnp.float32)
    hs_ref[...] = hs.astype(jnp.bfloat16)
    hd_ref[...] = hd.astype(jnp.bfloat16)


def _tc_matmuls(feat_pad, wt, wd, n_pad):
    bm = 512
    grid = (n_pad // bm,)
    return pl.pallas_call(
        _matmul_body,
        grid=grid,
        in_specs=[
            pl.BlockSpec((bm, D), lambda i: (i, 0)),
            pl.BlockSpec((D, D), lambda i: (0, 0)),
            pl.BlockSpec((D, D), lambda i: (0, 0)),
        ],
        out_specs=[
            pl.BlockSpec((bm, D), lambda i: (i, 0)),
            pl.BlockSpec((bm, D), lambda i: (i, 0)),
        ],
        out_shape=[
            jax.ShapeDtypeStruct((n_pad, D), jnp.bfloat16),
            jax.ShapeDtypeStruct((n_pad, D), jnp.bfloat16),
        ],
    )(feat_pad, wt, wd)


def _make_sc_kernel(n_pad):
    nch = n_pad // CH
    mesh = plsc.VectorSubcoreMesh(core_axis_name="c", subcore_axis_name="s")

    @functools.partial(
        pl.kernel,
        out_type=jax.ShapeDtypeStruct((D2, n_pad), jnp.int32),
        mesh=mesh,
        compiler_params=pltpu.CompilerParams(
            use_tc_tiling_on_sc=False, needs_layout_passes=False),
        scratch_types=[
            pltpu.VMEM((CPW, n_pad), jnp.int32),
            pltpu.VMEM((CPW, n_pad), jnp.int32),
            pltpu.VMEM((2, K, CH), jnp.int32),
            pltpu.VMEM((2, CPW, CH), jnp.int32),
            pltpu.SemaphoreType.DMA,
            pltpu.SemaphoreType.DMA,
            pltpu.SemaphoreType.DMA,
            pltpu.SemaphoreType.DMA,
        ],
    )
    def sc_kernel(hsrc_hbm, hdst_hbm, idx_hbm, out_hbm,
                  tab_v, hdst_v, idx_v, out_v, isem0, isem1, osem0, osem1):
        wid = lax.axis_index("s") * NC + lax.axis_index("c")
        col0 = wid * CPW
        pltpu.sync_copy(hsrc_hbm.at[pl.ds(col0, CPW)], tab_v)
        pltpu.sync_copy(hdst_hbm.at[pl.ds(col0, CPW)], hdst_v)
        isems = (isem0, isem1)
        osems = (osem0, osem1)

        def idx_start(ch, buf):
            pltpu.async_copy(idx_hbm.at[:, pl.ds(ch * CH, CH)],
                             idx_v.at[buf], isems[buf])

        def idx_wait(ch, buf):
            pltpu.make_async_copy(idx_hbm.at[:, pl.ds(ch * CH, CH)],
                                  idx_v.at[buf], isems[buf]).wait()

        def out_start(ch, buf):
            pltpu.async_copy(
                out_v.at[buf],
                out_hbm.at[pl.ds(col0, CPW), pl.ds(ch * CH, CH)],
                osems[buf])

        def out_wait(ch, buf):
            pltpu.make_async_copy(
                out_v.at[buf],
                out_hbm.at[pl.ds(col0, CPW), pl.ds(ch * CH, CH)],
                osems[buf]).wait()

        def chunk_compute(ch, buf):
            def group(g, carry):
                gsl = pl.ds(g * L, L)
                hsl = pl.ds(ch * CH + g * L, L)
                accs = []
                for col in range(CPW):
                    iv = idx_v[buf, 0, gsl]
                    x = plsc.load_gather(tab_v.at[col], [iv])
                    accs.append(plsc.bitcast(x, jnp.bfloat16))
                for j in range(1, K):
                    iv = idx_v[buf, j, gsl]
                    for col in range(CPW):
                        x = plsc.load_gather(tab_v.at[col], [iv])
                        accs[col] = jnp.maximum(
                            accs[col], plsc.bitcast(x, jnp.bfloat16))
                for col in range(CPW):
                    hd = plsc.bitcast(hdst_v[col, hsl], jnp.bfloat16)
                    out_v[buf, col, gsl] = plsc.bitcast(
                        accs[col] + hd, jnp.int32)
                return carry

            lax.fori_loop(0, CH // L, group, 0)

        idx_start(0, 0)
        for ch in range(nch):
            buf = ch % 2
            if ch + 1 < nch:
                idx_start(ch + 1, 1 - buf)
            idx_wait(ch, buf)
            if ch >= 2:
                out_wait(ch - 2, buf)
            chunk_compute(ch, buf)
            out_start(ch, buf)
        out_wait(nch - 2, nch % 2)
        out_wait(nch - 1, (nch - 1) % 2)

    return sc_kernel


def kernel(k, src_ind, feat, W_theta, W_phi):
    n = feat.shape[0]
    n_pad = -(-n // CH) * CH
    feat_pad = jnp.pad(feat, ((0, n_pad - n), (0, 0)))
    wt = W_theta.T
    wd = (W_phi - W_theta).T
    h_src, h_dst = _tc_matmuls(feat_pad, wt, wd, n_pad)

    idx_t = jnp.pad(src_ind.astype(jnp.int32),
                    ((0, n_pad - n), (0, 0))).T

    def pack_t(x):
        return lax.bitcast_convert_type(
            x.reshape(n_pad, D2, 2), jnp.int32).T

    out = _make_sc_kernel(n_pad)(pack_t(h_src), pack_t(h_dst), idx_t)
    out_bf = lax.bitcast_convert_type(out, jnp.bfloat16)
    out_f = out_bf.transpose(1, 0, 2).reshape(n_pad, D)
    return out_f[:n].astype(jnp.float32)
